# baseline (speedup 1.0000x reference)
import functools

import jax
import jax.numpy as jnp
import numpy as np
from jax import lax
from jax.experimental import pallas as pl
from jax.experimental.pallas import tpu as pltpu
from jax.experimental.pallas import tpu_sc as plsc

_PERM = np.arange(256).reshape(8, 2, 16).transpose(0, 2, 1).reshape(256)


def _nodeproj_body(x_ref, wd_ref, ws_ref, b_ref, pd_ref, ps_ref):
    xv = x_ref[...]
    pd = jnp.dot(xv, wd_ref[...], preferred_element_type=jnp.float32)
    pd_ref[...] = (pd + b_ref[...]).astype(jnp.bfloat16)
    ps = jnp.dot(xv, ws_ref[...], preferred_element_type=jnp.float32)
    ps_ref[...] = ps.astype(jnp.bfloat16)


def _nodeproj(x, wd, ws, bias):
    n, f = x.shape
    f2 = wd.shape[1]
    bn = 2000
    return pl.pallas_call(
        _nodeproj_body,
        grid=(n // bn,),
        in_specs=[
            pl.BlockSpec((bn, f), lambda i: (i, 0)),
            pl.BlockSpec((f, f2), lambda i: (0, 0)),
            pl.BlockSpec((f, f2), lambda i: (0, 0)),
            pl.BlockSpec((1, f2), lambda i: (0, 0)),
        ],
        out_specs=[
            pl.BlockSpec((bn, f2), lambda i: (i, 0)),
            pl.BlockSpec((bn, f2), lambda i: (i, 0)),
        ],
        out_shape=[jax.ShapeDtypeStruct((n, f2), jnp.bfloat16)] * 2,
    )(x, wd, ws, bias)


def _edgeproj_body(ea_ref, we_ref, ep_ref):
    ep = jnp.dot(ea_ref[...], we_ref[...], preferred_element_type=jnp.float32)
    ep_ref[...] = ep.astype(jnp.bfloat16)


def _edgeproj(edge_attr, we):
    e, d = edge_attr.shape
    f2 = we.shape[1]
    be = 3200
    return pl.pallas_call(
        _edgeproj_body,
        grid=(e // be,),
        in_specs=[
            pl.BlockSpec((be, d), lambda i: (i, 0)),
            pl.BlockSpec((d, f2), lambda i: (0, 0)),
        ],
        out_specs=pl.BlockSpec((be, f2), lambda i: (i, 0)),
        out_shape=jax.ShapeDtypeStruct((e, f2), jnp.bfloat16),
    )(edge_attr, we)


_C = 40
_STG = 32


def _UNPK(v):
    a = lax.bitcast_convert_type(jnp.left_shift(v, 16), jnp.float32)
    b = lax.bitcast_convert_type(
        jnp.bitwise_and(v, jnp.int32(-65536)), jnp.float32
    )
    return a, b


def _compute_chunk(pdv, psv, epv, msgv):
    @plsc.parallel_loop(0, _C, unroll=2)
    def _row(r):
        for b in range(4):
            gsl = pl.ds(16 * b, 16)
            csl = pl.ds(64 + 16 * b, 16)
            pg0, pg1 = _UNPK(pdv[r, gsl])
            sg0, sg1 = _UNPK(psv[r, gsl])
            eg0, eg1 = _UNPK(epv[r, gsl])
            pc0, pc1 = _UNPK(pdv[r, csl])
            sc0, sc1 = _UNPK(psv[r, csl])
            ec0, ec1 = _UNPK(epv[r, csl])
            for q, g, cz in (
                (2 * b, pg0 + sg0 + eg0, pc0 + sc0 + ec0),
                (2 * b + 1, pg1 + sg1 + eg1, pc1 + sc1 + ec1),
            ):
                eg = jnp.exp(-g)
                u = jnp.exp(-jnp.abs(cz))
                u2 = u + 2.0
                t = u / u2
                t2 = t * t
                poly = 1.0 + t2 * (0.33333334 + t2 * (0.2 + t2 * 0.14285715))
                num = jnp.maximum(cz, 0.0) * u2 + 2.0 * u * poly
                den = u2 * (1.0 + eg)
                msgv[r, pl.ds(16 * q, 16)] = num / den


def _sc_body(
    dst_hbm, src_hbm, pd_hbm, ps_hbm, ep_hbm,
    agg_out,
    dsti0, srci0, dsts0, pdv0, psv0, epv0, msgv0,
    dsti1, srci1, dsts1, pdv1, psv1, epv1, msgv1,
    stg, aggsh,
    spd0, sps0, sep0, ssc0, spd1, sps1, sep1, ssc1,
):
    npad = aggsh.shape[0]
    e = dst_hbm.shape[0]
    cid = lax.axis_index("c")
    sid = lax.axis_index("s")
    epc = e // 2
    epw = epc // 16
    nchunk = epw // _C
    rpt = npad // 16
    nstg = rpt // _STG

    sets = (
        (dsti0, srci0, dsts0, pdv0, psv0, epv0, msgv0, spd0, sps0, sep0, ssc0),
        (dsti1, srci1, dsts1, pdv1, psv1, epv1, msgv1, spd1, sps1, sep1, ssc1),
    )

    zero16 = jnp.zeros((16,), jnp.float32)

    def _zrow(r, carry):
        for j in range(8):
            stg[r, pl.ds(16 * j, 16)] = zero16
        return carry

    lax.fori_loop(0, _STG, _zrow, 0)

    def _zcp(k, carry):
        off = sid * rpt + k * _STG
        pltpu.sync_copy(stg, aggsh.at[pl.ds(off, _STG)])
        return carry

    lax.fori_loop(0, nstg, _zcp, 0)

    plsc.subcore_barrier()

    ebase = cid * epc + sid * epw

    def _load_idx(eb, s):
        dsti, srci = sets[s][0], sets[s][1]
        pltpu.sync_copy(dst_hbm.at[pl.ds(eb, _C)], dsti)
        pltpu.sync_copy(src_hbm.at[pl.ds(eb, _C)], srci)

    def _issue_gathers(eb, s):
        dsti, srci, _, pdv, psv, epv, _, spd, sps, sep, _ = sets[s]
        pltpu.async_copy(pd_hbm.at[dsti], pdv, spd)
        pltpu.async_copy(ps_hbm.at[srci], psv, sps)
        pltpu.async_copy(ep_hbm.at[pl.ds(eb, _C)], epv, sep)

    def _wait_gathers(s):
        dsti, srci, _, pdv, psv, epv, _, spd, sps, sep, _ = sets[s]
        pltpu.make_async_copy(pd_hbm.at[dsti], pdv, spd).wait()
        pltpu.make_async_copy(ps_hbm.at[srci], psv, sps).wait()
        pltpu.make_async_copy(ep_hbm.at[pl.ds(0, _C)], epv, sep).wait()

    def _wait_scatter(s):
        _, _, dsts, _, _, _, msgv, _, _, _, ssc = sets[s]
        pltpu.make_async_copy(msgv, aggsh.at[dsts], ssc).wait()

    _load_idx(ebase, 0)
    _issue_gathers(ebase, 0)

    def _pair(kk, carry):
        for s in range(2):
            o = 1 - s
            ci = 2 * kk + s
            dsti, srci, dsts, pdv, psv, epv, msgv, spd, sps, sep, ssc = sets[s]
            _wait_gathers(s)
            @pl.when(ci > 0)
            def _():
                _wait_scatter(o)

            ebn = ebase + jnp.minimum(ci + 1, nchunk - 1) * _C
            _load_idx(ebn, o)
            _issue_gathers(ebn, o)
            _compute_chunk(pdv, psv, epv, msgv)
            for off in (0, 16, 24):
                dsts[pl.ds(off, 16)] = dsti[pl.ds(off, 16)]
            pltpu.async_copy(msgv, aggsh.at[dsts], ssc, add=True)
        return carry

    lax.fori_loop(0, nchunk // 2, _pair, 0)

    _wait_scatter(1)
    _wait_gathers(0)

    plsc.subcore_barrier()

    def _rd(k, carry):
        off = sid * rpt + k * _STG
        pltpu.sync_copy(aggsh.at[pl.ds(off, _STG)], stg)
        pltpu.sync_copy(stg, agg_out.at[cid, pl.ds(off, _STG)])
        return carry

    lax.fori_loop(0, nstg, _rd, 0)


def _sc_edge_pass(dst, src, pd, ps, ep, npad):
    mesh = plsc.VectorSubcoreMesh(core_axis_name="c", subcore_axis_name="s")
    bufset = [
        pltpu.VMEM((_C,), jnp.int32),
        pltpu.VMEM((_C,), jnp.int32),
        pltpu.VMEM((_C,), jnp.int32),
        pltpu.VMEM((_C, 128), jnp.int32),
        pltpu.VMEM((_C, 128), jnp.int32),
        pltpu.VMEM((_C, 128), jnp.int32),
        pltpu.VMEM((_C, 128), jnp.float32),
    ]
    kern = pl.kernel(
        _sc_body,
        mesh=mesh,
        out_type=jax.ShapeDtypeStruct((2, npad, 128), jnp.float32),
        scratch_types=bufset + bufset + [
            pltpu.VMEM((_STG, 128), jnp.float32),
            pltpu.VMEM_SHARED((npad, 128), jnp.float32),
        ] + [pltpu.SemaphoreType.DMA] * 8,
    )
    return kern(dst, src, pd, ps, ep)


_CC = 80


def _sc_count_body(dst_hbm, cnt_out, dsti, onesv, stg16, cntsh):
    npad = cntsh.shape[0]
    e = dst_hbm.shape[0]
    cid = lax.axis_index("c")
    sid = lax.axis_index("s")
    epc = e // 2
    epw = epc // 16
    nchunk = epw // _CC
    rpt = npad // 16
    nstg = rpt // _STG

    zero16 = jnp.zeros((16,), jnp.float32)
    lanes = lax.iota(jnp.int32, 16)
    onerow = jnp.where(lanes == 0, 1.0, 0.0).astype(jnp.float32)

    def _init(r, carry):
        for j in range(8):
            stg16[r, pl.ds(16 * j, 16)] = zero16
        return carry

    lax.fori_loop(0, _STG, _init, 0)

    def _orow(r, carry):
        onesv[r, pl.ds(0, 16)] = onerow
        for j in range(1, 8):
            onesv[r, pl.ds(16 * j, 16)] = zero16
        return carry

    lax.fori_loop(0, _CC, _orow, 0)

    def _zcp(k, carry):
        off = sid * rpt + k * _STG
        pltpu.sync_copy(stg16, cntsh.at[pl.ds(off, _STG)])
        return carry

    lax.fori_loop(0, nstg, _zcp, 0)

    plsc.subcore_barrier()

    ebase = cid * epc + sid * epw

    def _chunk(ci, carry):
        eb = ebase + ci * _CC
        pltpu.sync_copy(dst_hbm.at[pl.ds(eb, _CC)], dsti)
        pltpu.sync_copy(onesv, cntsh.at[dsti], add=True)
        return carry

    lax.fori_loop(0, nchunk, _chunk, 0)

    plsc.subcore_barrier()

    def _rd(k, carry):
        off = sid * rpt + k * _STG
        pltpu.sync_copy(cntsh.at[pl.ds(off, _STG)], stg16)
        pltpu.sync_copy(stg16, cnt_out.at[cid, pl.ds(off, _STG)])
        return carry

    lax.fori_loop(0, nstg, _rd, 0)


def _sc_count_pass(dst, npad):
    mesh = plsc.VectorSubcoreMesh(core_axis_name="c", subcore_axis_name="s")
    kern = pl.kernel(
        _sc_count_body,
        mesh=mesh,
        out_type=jax.ShapeDtypeStruct((2, npad, 128), jnp.float32),
        scratch_types=[
            pltpu.VMEM((_CC,), jnp.int32),
            pltpu.VMEM((_CC, 128), jnp.float32),
            pltpu.VMEM((_STG, 128), jnp.float32),
            pltpu.VMEM_SHARED((npad, 128), jnp.float32),
        ],
    )
    return kern(dst)


def _final_body(x_ref, aggp_ref, cntp_ref, g_ref, b_ref, out_ref):
    n = x_ref.shape[0]
    agg = (aggp_ref[0] + aggp_ref[1])[:n]
    cnt = (cntp_ref[0] + cntp_ref[1])[:n]
    c = cnt[:, 0:1]
    out = x_ref[...] + agg / jnp.maximum(c, 1.0)
    m = jnp.mean(out, axis=0, keepdims=True)
    d = out - m
    v = jnp.mean(d * d, axis=0, keepdims=True)
    out_ref[...] = d * lax.rsqrt(v + 1e-5) * g_ref[...] + b_ref[...]


def _final(x, aggp, cntp, gamma, beta):
    n, f = x.shape
    return pl.pallas_call(
        _final_body,
        out_shape=jax.ShapeDtypeStruct((n, f), jnp.float32),
    )(x, aggp, cntp, gamma, beta)


def kernel(x, edge_index, edge_attr, Wf, bf, Ws, bs, gamma, beta):
    n, f = x.shape
    src = edge_index[0]
    dst = edge_index[1]
    perm = jnp.asarray(_PERM)
    wd = jnp.concatenate([Wf[:f], Ws[:f]], axis=1)[:, perm]
    wsrc = jnp.concatenate([Wf[f:2 * f], Ws[f:2 * f]], axis=1)[:, perm]
    we = jnp.concatenate([Wf[2 * f:], Ws[2 * f:]], axis=1)[:, perm]
    bias = jnp.concatenate([bf, bs])[perm].reshape(1, 2 * f)
    npad = ((n + 16 * _STG - 1) // (16 * _STG)) * (16 * _STG)
    pd, psrc = _nodeproj(x, wd, wsrc, bias)
    ep = _edgeproj(edge_attr, we)
    pd3 = lax.bitcast_convert_type(pd.reshape(n, 128, 2), jnp.int32)
    ps3 = lax.bitcast_convert_type(psrc.reshape(n, 128, 2), jnp.int32)
    ep3 = lax.bitcast_convert_type(
        ep.reshape(edge_attr.shape[0], 128, 2), jnp.int32
    )
    cntp = _sc_count_pass(dst, npad)
    aggp = _sc_edge_pass(dst, src, pd3, ps3, ep3, npad)
    return _final(
        x, aggp, cntp, gamma.reshape(1, f), beta.reshape(1, f)
    )


# speedup vs baseline: 2.2939x; 2.2939x over previous
import functools

import jax
import jax.numpy as jnp
import numpy as np
from jax import lax
from jax.experimental import pallas as pl
from jax.experimental.pallas import tpu as pltpu
from jax.experimental.pallas import tpu_sc as plsc

_PERM = np.arange(256).reshape(8, 2, 16).transpose(0, 2, 1).reshape(256)


def _pack_i32(a, b):
    au = lax.bitcast_convert_type(a.astype(jnp.bfloat16), jnp.uint16)
    bu = lax.bitcast_convert_type(b.astype(jnp.bfloat16), jnp.uint16)
    return (au.astype(jnp.int32) | jnp.left_shift(bu.astype(jnp.int32), 16))


def _nodeproj_body(x_ref, wdl_ref, wdh_ref, wsl_ref, wsh_ref, bl_ref, bh_ref,
                   pd_ref, ps_ref):
    xv = x_ref[...]
    pdl = jnp.dot(xv, wdl_ref[...], preferred_element_type=jnp.float32)
    pdh = jnp.dot(xv, wdh_ref[...], preferred_element_type=jnp.float32)
    pd_ref[...] = _pack_i32(pdl + bl_ref[...], pdh + bh_ref[...])
    psl = jnp.dot(xv, wsl_ref[...], preferred_element_type=jnp.float32)
    psh = jnp.dot(xv, wsh_ref[...], preferred_element_type=jnp.float32)
    ps_ref[...] = _pack_i32(psl, psh)


def _nodeproj(x, wdl, wdh, wsl, wsh, bl, bh):
    n, f = x.shape
    bn = 2000
    wspec = pl.BlockSpec((f, 128), lambda i: (0, 0))
    bspec = pl.BlockSpec((1, 128), lambda i: (0, 0))
    return pl.pallas_call(
        _nodeproj_body,
        grid=(n // bn,),
        in_specs=[pl.BlockSpec((bn, f), lambda i: (i, 0)),
                  wspec, wspec, wspec, wspec, bspec, bspec],
        out_specs=[
            pl.BlockSpec((bn, 128), lambda i: (i, 0)),
            pl.BlockSpec((bn, 128), lambda i: (i, 0)),
        ],
        out_shape=[jax.ShapeDtypeStruct((n, 128), jnp.int32)] * 2,
    )(x, wdl, wdh, wsl, wsh, bl, bh)


def _edgeproj_body(ea_ref, wel_ref, weh_ref, ep_ref):
    eav = ea_ref[...]
    epl = jnp.dot(eav, wel_ref[...], preferred_element_type=jnp.float32)
    eph = jnp.dot(eav, weh_ref[...], preferred_element_type=jnp.float32)
    ep_ref[...] = _pack_i32(epl, eph)


def _edgeproj(edge_attr, wel, weh):
    e, d = edge_attr.shape
    be = 3200
    wspec = pl.BlockSpec((d, 128), lambda i: (0, 0))
    return pl.pallas_call(
        _edgeproj_body,
        grid=(e // be,),
        in_specs=[pl.BlockSpec((be, d), lambda i: (i, 0)), wspec, wspec],
        out_specs=pl.BlockSpec((be, 128), lambda i: (i, 0)),
        out_shape=jax.ShapeDtypeStruct((e, 128), jnp.int32),
    )(edge_attr, wel, weh)


_C = 40
_STG = 32


def _UNPK(v):
    a = lax.bitcast_convert_type(jnp.left_shift(v, 16), jnp.float32)
    b = lax.bitcast_convert_type(
        jnp.bitwise_and(v, jnp.int32(-65536)), jnp.float32
    )
    return a, b


def _compute_chunk(pdv, psv, epv, msgv):
    @plsc.parallel_loop(0, _C, unroll=2)
    def _row(r):
        for b in range(4):
            gsl = pl.ds(16 * b, 16)
            csl = pl.ds(64 + 16 * b, 16)
            pg0, pg1 = _UNPK(pdv[r, gsl])
            sg0, sg1 = _UNPK(psv[r, gsl])
            eg0, eg1 = _UNPK(epv[r, gsl])
            pc0, pc1 = _UNPK(pdv[r, csl])
            sc0, sc1 = _UNPK(psv[r, csl])
            ec0, ec1 = _UNPK(epv[r, csl])
            for q, g, cz in (
                (2 * b, pg0 + sg0 + eg0, pc0 + sc0 + ec0),
                (2 * b + 1, pg1 + sg1 + eg1, pc1 + sc1 + ec1),
            ):
                eg = jnp.exp(-g)
                u = jnp.exp(-jnp.abs(cz))
                u2 = u + 2.0
                t = u / u2
                t2 = t * t
                poly = 1.0 + t2 * (0.33333334 + t2 * (0.2 + t2 * 0.14285715))
                num = jnp.maximum(cz, 0.0) * u2 + 2.0 * u * poly
                den = u2 * (1.0 + eg)
                msgv[r, pl.ds(16 * q, 16)] = num / den


def _sc_body(
    dst_hbm, src_hbm, pd_hbm, ps_hbm, ep_hbm,
    agg_out,
    dsti0, srci0, dsts0, pdv0, psv0, epv0, msgv0,
    dsti1, srci1, dsts1, pdv1, psv1, epv1, msgv1,
    stg, aggsh,
    spd0, sps0, sep0, ssc0, spd1, sps1, sep1, ssc1,
):
    npad = aggsh.shape[0]
    e = dst_hbm.shape[0]
    cid = lax.axis_index("c")
    sid = lax.axis_index("s")
    epc = e // 2
    epw = epc // 16
    nchunk = epw // _C
    rpt = npad // 16
    nstg = rpt // _STG

    sets = (
        (dsti0, srci0, dsts0, pdv0, psv0, epv0, msgv0, spd0, sps0, sep0, ssc0),
        (dsti1, srci1, dsts1, pdv1, psv1, epv1, msgv1, spd1, sps1, sep1, ssc1),
    )

    zero16 = jnp.zeros((16,), jnp.float32)

    def _zrow(r, carry):
        for j in range(8):
            stg[r, pl.ds(16 * j, 16)] = zero16
        return carry

    lax.fori_loop(0, _STG, _zrow, 0)

    def _zcp(k, carry):
        off = sid * rpt + k * _STG
        pltpu.sync_copy(stg, aggsh.at[pl.ds(off, _STG)])
        return carry

    lax.fori_loop(0, nstg, _zcp, 0)

    plsc.subcore_barrier()

    ebase = cid * epc + sid * epw

    def _load_idx(eb, s):
        dsti, srci = sets[s][0], sets[s][1]
        pltpu.sync_copy(dst_hbm.at[pl.ds(eb, _C)], dsti)
        pltpu.sync_copy(src_hbm.at[pl.ds(eb, _C)], srci)

    def _issue_gathers(eb, s):
        dsti, srci, _, pdv, psv, epv, _, spd, sps, sep, _ = sets[s]
        pltpu.async_copy(pd_hbm.at[dsti], pdv, spd)
        pltpu.async_copy(ps_hbm.at[srci], psv, sps)
        pltpu.async_copy(ep_hbm.at[pl.ds(eb, _C)], epv, sep)

    def _wait_gathers(s):
        dsti, srci, _, pdv, psv, epv, _, spd, sps, sep, _ = sets[s]
        pltpu.make_async_copy(pd_hbm.at[dsti], pdv, spd).wait()
        pltpu.make_async_copy(ps_hbm.at[srci], psv, sps).wait()
        pltpu.make_async_copy(ep_hbm.at[pl.ds(0, _C)], epv, sep).wait()

    def _wait_scatter(s):
        _, _, dsts, _, _, _, msgv, _, _, _, ssc = sets[s]
        pltpu.make_async_copy(msgv, aggsh.at[dsts], ssc).wait()

    _load_idx(ebase, 0)
    _issue_gathers(ebase, 0)

    def _pair(kk, carry):
        for s in range(2):
            o = 1 - s
            ci = 2 * kk + s
            dsti, srci, dsts, pdv, psv, epv, msgv, spd, sps, sep, ssc = sets[s]
            _wait_gathers(s)
            @pl.when(ci > 0)
            def _():
                _wait_scatter(o)

            ebn = ebase + jnp.minimum(ci + 1, nchunk - 1) * _C
            _load_idx(ebn, o)
            _issue_gathers(ebn, o)
            _compute_chunk(pdv, psv, epv, msgv)
            for off in (0, 16, 24):
                dsts[pl.ds(off, 16)] = dsti[pl.ds(off, 16)]
            pltpu.async_copy(msgv, aggsh.at[dsts], ssc, add=True)
        return carry

    lax.fori_loop(0, nchunk // 2, _pair, 0)

    _wait_scatter(1)
    _wait_gathers(0)

    plsc.subcore_barrier()

    def _rd(k, carry):
        off = sid * rpt + k * _STG
        pltpu.sync_copy(aggsh.at[pl.ds(off, _STG)], stg)
        pltpu.sync_copy(stg, agg_out.at[cid, pl.ds(off, _STG)])
        return carry

    lax.fori_loop(0, nstg, _rd, 0)


def _sc_edge_pass(dst, src, pd, ps, ep, npad):
    mesh = plsc.VectorSubcoreMesh(core_axis_name="c", subcore_axis_name="s")
    bufset = [
        pltpu.VMEM((_C,), jnp.int32),
        pltpu.VMEM((_C,), jnp.int32),
        pltpu.VMEM((_C,), jnp.int32),
        pltpu.VMEM((_C, 128), jnp.int32),
        pltpu.VMEM((_C, 128), jnp.int32),
        pltpu.VMEM((_C, 128), jnp.int32),
        pltpu.VMEM((_C, 128), jnp.float32),
    ]
    kern = pl.kernel(
        _sc_body,
        mesh=mesh,
        out_type=jax.ShapeDtypeStruct((2, npad, 128), jnp.float32),
        scratch_types=bufset + bufset + [
            pltpu.VMEM((_STG, 128), jnp.float32),
            pltpu.VMEM_SHARED((npad, 128), jnp.float32),
        ] + [pltpu.SemaphoreType.DMA] * 8,
    )
    return kern(dst, src, pd, ps, ep)


_CC = 80


def _sc_count_body(dst_hbm, cnt_out, dsti, onesv, stg16, cntsh):
    npad = cntsh.shape[0]
    e = dst_hbm.shape[0]
    cid = lax.axis_index("c")
    sid = lax.axis_index("s")
    epc = e // 2
    epw = epc // 16
    nchunk = epw // _CC
    rpt = npad // 16
    nstg = rpt // _STG

    zero16 = jnp.zeros((16,), jnp.float32)
    lanes = lax.iota(jnp.int32, 16)
    onerow = jnp.where(lanes == 0, 1.0, 0.0).astype(jnp.float32)

    def _init(r, carry):
        for j in range(8):
            stg16[r, pl.ds(16 * j, 16)] = zero16
        return carry

    lax.fori_loop(0, _STG, _init, 0)

    def _orow(r, carry):
        onesv[r, pl.ds(0, 16)] = onerow
        for j in range(1, 8):
            onesv[r, pl.ds(16 * j, 16)] = zero16
        return carry

    lax.fori_loop(0, _CC, _orow, 0)

    def _zcp(k, carry):
        off = sid * rpt + k * _STG
        pltpu.sync_copy(stg16, cntsh.at[pl.ds(off, _STG)])
        return carry

    lax.fori_loop(0, nstg, _zcp, 0)

    plsc.subcore_barrier()

    ebase = cid * epc + sid * epw

    def _chunk(ci, carry):
        eb = ebase + ci * _CC
        pltpu.sync_copy(dst_hbm.at[pl.ds(eb, _CC)], dsti)
        pltpu.sync_copy(onesv, cntsh.at[dsti], add=True)
        return carry

    lax.fori_loop(0, nchunk, _chunk, 0)

    plsc.subcore_barrier()

    def _rd(k, carry):
        off = sid * rpt + k * _STG
        pltpu.sync_copy(cntsh.at[pl.ds(off, _STG)], stg16)
        pltpu.sync_copy(stg16, cnt_out.at[cid, pl.ds(off, _STG)])
        return carry

    lax.fori_loop(0, nstg, _rd, 0)


def _sc_count_pass(dst, npad):
    mesh = plsc.VectorSubcoreMesh(core_axis_name="c", subcore_axis_name="s")
    kern = pl.kernel(
        _sc_count_body,
        mesh=mesh,
        out_type=jax.ShapeDtypeStruct((2, npad, 128), jnp.float32),
        scratch_types=[
            pltpu.VMEM((_CC,), jnp.int32),
            pltpu.VMEM((_CC, 128), jnp.float32),
            pltpu.VMEM((_STG, 128), jnp.float32),
            pltpu.VMEM_SHARED((npad, 128), jnp.float32),
        ],
    )
    return kern(dst)


def _final_body(x_ref, aggp_ref, cntp_ref, g_ref, b_ref, out_ref):
    n = x_ref.shape[0]
    agg = (aggp_ref[0] + aggp_ref[1])[:n]
    cnt = (cntp_ref[0] + cntp_ref[1])[:n]
    c = cnt[:, 0:1]
    out = x_ref[...] + agg / jnp.maximum(c, 1.0)
    m = jnp.mean(out, axis=0, keepdims=True)
    d = out - m
    v = jnp.mean(d * d, axis=0, keepdims=True)
    out_ref[...] = d * lax.rsqrt(v + 1e-5) * g_ref[...] + b_ref[...]


def _final(x, aggp, cntp, gamma, beta):
    n, f = x.shape
    return pl.pallas_call(
        _final_body,
        out_shape=jax.ShapeDtypeStruct((n, f), jnp.float32),
    )(x, aggp, cntp, gamma, beta)


def kernel(x, edge_index, edge_attr, Wf, bf, Ws, bs, gamma, beta):
    n, f = x.shape
    src = edge_index[0]
    dst = edge_index[1]
    cols_lo = jnp.asarray(_PERM[0::2])
    cols_hi = jnp.asarray(_PERM[1::2])
    wd = jnp.concatenate([Wf[:f], Ws[:f]], axis=1)
    wsrc = jnp.concatenate([Wf[f:2 * f], Ws[f:2 * f]], axis=1)
    we = jnp.concatenate([Wf[2 * f:], Ws[2 * f:]], axis=1)
    bias = jnp.concatenate([bf, bs])
    npad = ((n + 16 * _STG - 1) // (16 * _STG)) * (16 * _STG)
    pd3, ps3 = _nodeproj(
        x, wd[:, cols_lo], wd[:, cols_hi], wsrc[:, cols_lo], wsrc[:, cols_hi],
        bias[cols_lo].reshape(1, 128), bias[cols_hi].reshape(1, 128),
    )
    ep3 = _edgeproj(edge_attr, we[:, cols_lo], we[:, cols_hi])
    cntp = _sc_count_pass(dst, npad)
    aggp = _sc_edge_pass(dst, src, pd3, ps3, ep3, npad)
    return _final(
        x, aggp, cntp, gamma.reshape(1, f), beta.reshape(1, f)
    )
